# baseline (speedup 1.0000x reference)
import functools

import jax
import jax.numpy as jnp
from jax import lax
from jax.experimental import pallas as pl
from jax.experimental.pallas import tpu as pltpu
from jax.experimental.pallas import tpu_sc as plsc

N = 10000
E = 320000
H = 128
G = 256
C = 10

NC = 2
NS = 16
NW = NC * NS

CHUNK = 128
CPT = (((E + NW * CHUNK - 1) // (NW * CHUNK)) + 7) // 8 * 8
EPAD = NW * CHUNK * CPT
ZROWS = ((N // NS) // 8 + 1) * 8
NACC = ZROWS * NS


def _sc_mesh():
    return plsc.VectorSubcoreMesh(core_axis_name="c", subcore_axis_name="s",
                                  num_cores=NC, num_subcores=NS)


KG = 5
KS = 2
assert CPT % KG == 0 and CPT % KS == 0
NG = CPT // KG
NGS = CPT // KS


@functools.partial(
    pl.kernel,
    out_type=jax.ShapeDtypeStruct((EPAD, H), jnp.float32),
    mesh=_sc_mesh(),
    scratch_types=[
        pltpu.VMEM((CPT, CHUNK), jnp.int32),
        pltpu.VMEM((KG * CHUNK, H), jnp.float32),
        pltpu.SemaphoreType.DMA,
    ],
)
def _sc_gather(h_hbm, src_hbm, msgs_hbm, src_v, rows, gsem):
    cid = lax.axis_index("c")
    sid = lax.axis_index("s")
    wid = sid * NC + cid

    pltpu.sync_copy(src_hbm.at[pl.ds(wid * CPT, CPT)], src_v)

    def body(g, carry):
        j0 = g * KG
        for b in range(KG):
            pltpu.async_copy(h_hbm.at[src_v.at[j0 + b]],
                             rows.at[pl.ds(b * CHUNK, CHUNK)], gsem)
        for b in range(KG):
            pltpu.make_async_copy(h_hbm.at[src_v.at[0]],
                                  rows.at[pl.ds(b * CHUNK, CHUNK)], gsem).wait()
        pltpu.sync_copy(rows, msgs_hbm.at[pl.ds((wid * CPT + j0) * CHUNK,
                                                KG * CHUNK)])
        return carry

    lax.fori_loop(0, NG, body, 0)


@functools.partial(
    pl.kernel,
    out_type=jax.ShapeDtypeStruct((NC, NACC, H), jnp.float32),
    mesh=_sc_mesh(),
    scratch_types=[
        pltpu.VMEM((CPT, CHUNK), jnp.int32),
        pltpu.VMEM((KS * CHUNK, H), jnp.float32),
        pltpu.VMEM_SHARED((NACC, H), jnp.float32),
    ],
)
def _sc_scatter(msgs_hbm, dst_hbm, zeros_hbm, out_hbm, dst_v, rows, acc):
    cid = lax.axis_index("c")
    sid = lax.axis_index("s")
    wid = sid * NC + cid

    pltpu.sync_copy(zeros_hbm, acc.at[pl.ds(sid * ZROWS, ZROWS)])
    pltpu.sync_copy(dst_hbm.at[pl.ds(wid * CPT, CPT)], dst_v)
    plsc.subcore_barrier()

    def body(g, carry):
        j0 = g * KS
        pltpu.sync_copy(msgs_hbm.at[pl.ds((wid * CPT + j0) * CHUNK,
                                          KS * CHUNK)], rows)
        for b in range(KS):
            pltpu.sync_copy(rows.at[pl.ds(b * CHUNK, CHUNK)],
                            acc.at[dst_v.at[j0 + b]], add=True)
        return carry

    lax.fori_loop(0, NGS, body, 0)
    plsc.subcore_barrier()

    pltpu.sync_copy(acc.at[pl.ds(sid * ZROWS, ZROWS)],
                    out_hbm.at[cid, pl.ds(sid * ZROWS, ZROWS)])


ROWS = 1000
GRID = N // ROWS


def _mlp_body(h_ref, a0_ref, a1_ref, w1_ref, b1_ref, w2_ref, b2_ref, eps_ref,
              out_ref):
    z = (1.0 + eps_ref[0, 0]) * h_ref[...] + a0_ref[...] + a1_ref[...]
    z = jnp.dot(z, w1_ref[...], preferred_element_type=jnp.float32) + b1_ref[...]
    z = jnp.maximum(z, 0.0)
    z = jnp.dot(z, w2_ref[...], preferred_element_type=jnp.float32) + b2_ref[...]
    out_ref[...] = jnp.maximum(z, 0.0)


_row_spec = pl.BlockSpec((ROWS, H), lambda i: (i, 0))
_full_spec = pl.BlockSpec((H, H), lambda i: (0, 0))
_vec_spec = pl.BlockSpec((1, H), lambda i: (0, 0))
_scalar_spec = pl.BlockSpec((1, 1), lambda i: (0, 0))

_tc_mlp = pl.pallas_call(
    _mlp_body,
    grid=(GRID,),
    in_specs=[_row_spec, _row_spec, _row_spec, _full_spec, _vec_spec,
              _full_spec, _vec_spec, _scalar_spec],
    out_specs=_row_spec,
    out_shape=jax.ShapeDtypeStruct((N, H), jnp.float32),
)


def _pool_body(h_ref, batch_ref, wout_ref, bout_ref, out_ref, acc_ref):
    i = pl.program_id(0)

    @pl.when(i == 0)
    def _():
        acc_ref[...] = jnp.zeros_like(acc_ref)

    gids = lax.broadcasted_iota(jnp.int32, (ROWS, G), 1)
    onehot = (batch_ref[...] == gids).astype(jnp.float32)
    acc_ref[...] += lax.dot_general(
        onehot, h_ref[...], (((0,), (0,)), ((), ())),
        preferred_element_type=jnp.float32)

    @pl.when(i == GRID - 1)
    def _():
        out_ref[...] = (jnp.dot(acc_ref[...], wout_ref[...],
                                preferred_element_type=jnp.float32)
                        + bout_ref[...])


_tc_pool = pl.pallas_call(
    _pool_body,
    grid=(GRID,),
    in_specs=[_row_spec,
              pl.BlockSpec((ROWS, 1), lambda i: (i, 0)),
              pl.BlockSpec((H, C), lambda i: (0, 0)),
              pl.BlockSpec((1, C), lambda i: (0, 0))],
    out_specs=pl.BlockSpec((G, C), lambda i: (0, 0)),
    out_shape=jax.ShapeDtypeStruct((G, C), jnp.float32),
    scratch_shapes=[pltpu.VMEM((G, H), jnp.float32)],
)


def kernel(x, edge_index, batch, params):
    src = edge_index[0]
    dst = edge_index[1]
    pad = EPAD - E
    src2d = jnp.concatenate([src, jnp.zeros((pad,), jnp.int32)]).reshape(-1, CHUNK)
    dst2d = jnp.concatenate([dst, jnp.full((pad,), N, jnp.int32)]).reshape(-1, CHUNK)
    zeros = jnp.zeros((ZROWS, H), jnp.float32)

    h = x
    for l in range(3):
        msgs = _sc_gather(h, src2d)
        parts = _sc_scatter(msgs, dst2d, zeros)
        h = _tc_mlp(h, parts[0, :N], parts[1, :N],
                    params[f"W1_{l}"], params[f"b1_{l}"].reshape(1, H),
                    params[f"W2_{l}"], params[f"b2_{l}"].reshape(1, H),
                    params[f"eps_{l}"].reshape(1, 1))

    return _tc_pool(h, batch.reshape(N, 1), params["Wout"],
                    params["bout"].reshape(1, C))


# speedup vs baseline: 1.4243x; 1.4243x over previous
import functools

import jax
import jax.numpy as jnp
from jax import lax
from jax.experimental import pallas as pl
from jax.experimental.pallas import tpu as pltpu
from jax.experimental.pallas import tpu_sc as plsc

N = 10000
E = 320000
H = 128
G = 256
C = 10

NC = 2
NS = 16
NW = NC * NS

CHUNK = 128
CPT = (((E + NW * CHUNK - 1) // (NW * CHUNK)) + 7) // 8 * 8
EPAD = NW * CHUNK * CPT
ZROWS = ((N // NS) // 8 + 1) * 8
NACC = ZROWS * NS


def _sc_mesh():
    return plsc.VectorSubcoreMesh(core_axis_name="c", subcore_axis_name="s",
                                  num_cores=NC, num_subcores=NS)


NBUF = 2
assert CPT % NBUF == 0


@functools.partial(
    pl.kernel,
    out_type=jax.ShapeDtypeStruct((NC, NACC, H), jnp.float32),
    mesh=_sc_mesh(),
    scratch_types=[
        pltpu.VMEM((CPT, CHUNK), jnp.int32),
        pltpu.VMEM((CPT, CHUNK), jnp.int32),
        pltpu.VMEM((CHUNK, H), jnp.float32),
        pltpu.VMEM((CHUNK, H), jnp.float32),
        pltpu.VMEM_SHARED((NACC, H), jnp.float32),
        pltpu.SemaphoreType.DMA,
        pltpu.SemaphoreType.DMA,
    ],
)
def _sc_segsum(h_hbm, src_hbm, dst_hbm, zeros_hbm, out_hbm,
               src_v, dst_v, rows0, rows1, acc, gsem0, gsem1):
    rows = [rows0, rows1]
    gsem = [gsem0, gsem1]
    cid = lax.axis_index("c")
    sid = lax.axis_index("s")
    wid = sid * NC + cid

    pltpu.sync_copy(zeros_hbm, acc.at[pl.ds(sid * ZROWS, ZROWS)])

    pltpu.sync_copy(src_hbm.at[pl.ds(wid * CPT, CPT)], src_v)
    pltpu.sync_copy(dst_hbm.at[pl.ds(wid * CPT, CPT)], dst_v)
    plsc.subcore_barrier()

    def body(j, carry):
        pltpu.async_copy(h_hbm.at[src_v.at[j]], rows0, gsem0).wait()
        pltpu.sync_copy(rows0, acc.at[dst_v.at[j]], add=True)
        return carry

    lax.fori_loop(0, CPT, body, 0)
    plsc.subcore_barrier()

    pltpu.sync_copy(acc.at[pl.ds(sid * ZROWS, ZROWS)],
                    out_hbm.at[cid, pl.ds(sid * ZROWS, ZROWS)])


ROWS = 1000
GRID = N // ROWS


def _mlp_body(h_ref, a0_ref, a1_ref, w1_ref, b1_ref, w2_ref, b2_ref, eps_ref,
              out_ref):
    z = (1.0 + eps_ref[0, 0]) * h_ref[...] + a0_ref[...] + a1_ref[...]
    z = jnp.dot(z, w1_ref[...], preferred_element_type=jnp.float32) + b1_ref[...]
    z = jnp.maximum(z, 0.0)
    z = jnp.dot(z, w2_ref[...], preferred_element_type=jnp.float32) + b2_ref[...]
    out_ref[...] = jnp.maximum(z, 0.0)


_row_spec = pl.BlockSpec((ROWS, H), lambda i: (i, 0))
_full_spec = pl.BlockSpec((H, H), lambda i: (0, 0))
_vec_spec = pl.BlockSpec((1, H), lambda i: (0, 0))
_scalar_spec = pl.BlockSpec((1, 1), lambda i: (0, 0))

_tc_mlp = pl.pallas_call(
    _mlp_body,
    grid=(GRID,),
    in_specs=[_row_spec, _row_spec, _row_spec, _full_spec, _vec_spec,
              _full_spec, _vec_spec, _scalar_spec],
    out_specs=_row_spec,
    out_shape=jax.ShapeDtypeStruct((N, H), jnp.float32),
)


def _pool_body(h_ref, batch_ref, wout_ref, bout_ref, out_ref, acc_ref):
    i = pl.program_id(0)

    @pl.when(i == 0)
    def _():
        acc_ref[...] = jnp.zeros_like(acc_ref)

    gids = lax.broadcasted_iota(jnp.int32, (ROWS, G), 1)
    onehot = (batch_ref[...] == gids).astype(jnp.float32)
    acc_ref[...] += lax.dot_general(
        onehot, h_ref[...], (((0,), (0,)), ((), ())),
        preferred_element_type=jnp.float32)

    @pl.when(i == GRID - 1)
    def _():
        out_ref[...] = (jnp.dot(acc_ref[...], wout_ref[...],
                                preferred_element_type=jnp.float32)
                        + bout_ref[...])


_tc_pool = pl.pallas_call(
    _pool_body,
    grid=(GRID,),
    in_specs=[_row_spec,
              pl.BlockSpec((ROWS, 1), lambda i: (i, 0)),
              pl.BlockSpec((H, C), lambda i: (0, 0)),
              pl.BlockSpec((1, C), lambda i: (0, 0))],
    out_specs=pl.BlockSpec((G, C), lambda i: (0, 0)),
    out_shape=jax.ShapeDtypeStruct((G, C), jnp.float32),
    scratch_shapes=[pltpu.VMEM((G, H), jnp.float32)],
)


def kernel(x, edge_index, batch, params):
    src = edge_index[0]
    dst = edge_index[1]
    pad = EPAD - E
    src2d = jnp.concatenate([src, jnp.zeros((pad,), jnp.int32)]).reshape(-1, CHUNK)
    dst2d = jnp.concatenate([dst, jnp.full((pad,), N, jnp.int32)]).reshape(-1, CHUNK)
    zeros = jnp.zeros((ZROWS, H), jnp.float32)

    h = x
    for l in range(3):
        parts = _sc_segsum(h, src2d, dst2d, zeros)
        h = _tc_mlp(h, parts[0, :N], parts[1, :N],
                    params[f"W1_{l}"], params[f"b1_{l}"].reshape(1, H),
                    params[f"W2_{l}"], params[f"b2_{l}"].reshape(1, H),
                    params[f"eps_{l}"].reshape(1, 1))

    return _tc_pool(h, batch.reshape(N, 1), params["Wout"],
                    params["bout"].reshape(1, C))
